# Initial kernel scaffold; baseline (speedup 1.0000x reference)
import jax
import jax.numpy as jnp
from jax.experimental import pallas as pl


def kernel(x, W, b):
    raise NotImplementedError("write your pallas kernel here")


# speedup vs baseline: 3.4558x; 3.4558x over previous
import jax
import jax.numpy as jnp
from jax.experimental import pallas as pl
from jax.experimental.pallas import tpu as pltpu

_ROWS = 512
_K = 3


def _router_kernel(x_ref, w_ref, b_ref, mask_ref, prob_ref, imp_ref):
    logits = jnp.dot(x_ref[...], w_ref[...],
                     preferred_element_type=jnp.float32) + b_ref[...]

    m = jnp.max(logits, axis=-1, keepdims=True)
    e = jnp.exp(logits - m)
    prob = e / jnp.sum(e, axis=-1, keepdims=True)
    prob_ref[...] = prob

    @pl.when(pl.program_id(0) == 0)
    def _init():
        imp_ref[...] = jnp.zeros_like(imp_ref)

    imp_ref[...] += jnp.sum(prob, axis=0, keepdims=True)

    n_e = logits.shape[-1]
    cols = jax.lax.broadcasted_iota(jnp.int32, logits.shape, 1)
    work = logits
    mask = jnp.zeros_like(logits)
    for _ in range(_K):
        mx = jnp.max(work, axis=-1, keepdims=True)
        cand = jnp.where(work == mx, cols, n_e)
        sel = jnp.min(cand, axis=-1, keepdims=True)
        hit = cols == sel
        mask = mask + hit.astype(jnp.float32)
        work = jnp.where(hit, -jnp.inf, work)
    mask_ref[...] = mask


def kernel(x, W, b):
    x = x.reshape(x.shape[0], -1)
    n, d = x.shape
    n_e = W.shape[1]
    grid = n // _ROWS
    mask, prob, imp = pl.pallas_call(
        _router_kernel,
        grid=(grid,),
        in_specs=[
            pl.BlockSpec((_ROWS, d), lambda i: (i, 0)),
            pl.BlockSpec((d, n_e), lambda i: (0, 0)),
            pl.BlockSpec((1, n_e), lambda i: (0, 0)),
        ],
        out_specs=[
            pl.BlockSpec((_ROWS, n_e), lambda i: (i, 0)),
            pl.BlockSpec((_ROWS, n_e), lambda i: (i, 0)),
            pl.BlockSpec((1, n_e), lambda i: (0, 0)),
        ],
        out_shape=[
            jax.ShapeDtypeStruct((n, n_e), jnp.float32),
            jax.ShapeDtypeStruct((n, n_e), jnp.float32),
            jax.ShapeDtypeStruct((1, n_e), jnp.float32),
        ],
        compiler_params=pltpu.CompilerParams(
            dimension_semantics=("arbitrary",)),
    )(x, W, b.reshape(1, -1))
    imp = imp.reshape(-1)
    return mask, prob, imp, imp


# speedup vs baseline: 3.9377x; 1.1395x over previous
import jax
import jax.numpy as jnp
from jax.experimental import pallas as pl
from jax.experimental.pallas import tpu as pltpu

_ROWS = 1024
_K = 3


def _router_kernel(x_ref, w_ref, b_ref, mask_ref, prob_ref, imp_ref):
    logits = jnp.dot(x_ref[...], w_ref[...],
                     preferred_element_type=jnp.float32) + b_ref[...]

    m = jnp.max(logits, axis=-1, keepdims=True)
    e = jnp.exp(logits - m)
    prob = e / jnp.sum(e, axis=-1, keepdims=True)
    prob_ref[...] = prob

    @pl.when(pl.program_id(0) == 0)
    def _init():
        imp_ref[...] = jnp.zeros_like(imp_ref)

    imp_ref[...] += jnp.sum(prob, axis=0, keepdims=True)

    n_e = logits.shape[-1]
    cols = jax.lax.broadcasted_iota(jnp.int32, logits.shape, 1)
    work = logits
    mask = jnp.zeros_like(logits)
    for _ in range(_K):
        mx = jnp.max(work, axis=-1, keepdims=True)
        cand = jnp.where(work == mx, cols, n_e)
        sel = jnp.min(cand, axis=-1, keepdims=True)
        hit = cols == sel
        mask = mask + hit.astype(jnp.float32)
        work = jnp.where(hit, -jnp.inf, work)
    mask_ref[...] = mask


def kernel(x, W, b):
    x = x.reshape(x.shape[0], -1)
    n, d = x.shape
    n_e = W.shape[1]
    grid = n // _ROWS
    mask, prob, imp = pl.pallas_call(
        _router_kernel,
        grid=(grid,),
        in_specs=[
            pl.BlockSpec((_ROWS, d), lambda i: (i, 0)),
            pl.BlockSpec((d, n_e), lambda i: (0, 0)),
            pl.BlockSpec((1, n_e), lambda i: (0, 0)),
        ],
        out_specs=[
            pl.BlockSpec((_ROWS, n_e), lambda i: (i, 0)),
            pl.BlockSpec((_ROWS, n_e), lambda i: (i, 0)),
            pl.BlockSpec((1, n_e), lambda i: (0, 0)),
        ],
        out_shape=[
            jax.ShapeDtypeStruct((n, n_e), jnp.float32),
            jax.ShapeDtypeStruct((n, n_e), jnp.float32),
            jax.ShapeDtypeStruct((1, n_e), jnp.float32),
        ],
        compiler_params=pltpu.CompilerParams(
            dimension_semantics=("arbitrary",)),
    )(x, W, b.reshape(1, -1))
    imp = imp.reshape(-1)
    return mask, prob, imp, imp


# speedup vs baseline: 4.0660x; 1.0326x over previous
import jax
import jax.numpy as jnp
from jax.experimental import pallas as pl
from jax.experimental.pallas import tpu as pltpu

_ROWS = 2048
_K = 3


def _router_kernel(x_ref, w_ref, b_ref, mask_ref, prob_ref, imp_ref):
    logits = jnp.dot(x_ref[...], w_ref[...],
                     preferred_element_type=jnp.float32) + b_ref[...]

    m = jnp.max(logits, axis=-1, keepdims=True)
    e = jnp.exp(logits - m)
    prob = e / jnp.sum(e, axis=-1, keepdims=True)
    prob_ref[...] = prob

    @pl.when(pl.program_id(0) == 0)
    def _init():
        imp_ref[...] = jnp.zeros_like(imp_ref)

    imp_ref[...] += jnp.sum(prob, axis=0, keepdims=True)

    n_e = logits.shape[-1]
    cols = jax.lax.broadcasted_iota(jnp.int32, logits.shape, 1)
    work = logits
    mask = jnp.zeros_like(logits)
    for _ in range(_K):
        mx = jnp.max(work, axis=-1, keepdims=True)
        cand = jnp.where(work == mx, cols, n_e)
        sel = jnp.min(cand, axis=-1, keepdims=True)
        hit = cols == sel
        mask = mask + hit.astype(jnp.float32)
        work = jnp.where(hit, -jnp.inf, work)
    mask_ref[...] = mask


def kernel(x, W, b):
    x = x.reshape(x.shape[0], -1)
    n, d = x.shape
    n_e = W.shape[1]
    grid = n // _ROWS
    mask, prob, imp = pl.pallas_call(
        _router_kernel,
        grid=(grid,),
        in_specs=[
            pl.BlockSpec((_ROWS, d), lambda i: (i, 0)),
            pl.BlockSpec((d, n_e), lambda i: (0, 0)),
            pl.BlockSpec((1, n_e), lambda i: (0, 0)),
        ],
        out_specs=[
            pl.BlockSpec((_ROWS, n_e), lambda i: (i, 0)),
            pl.BlockSpec((_ROWS, n_e), lambda i: (i, 0)),
            pl.BlockSpec((1, n_e), lambda i: (0, 0)),
        ],
        out_shape=[
            jax.ShapeDtypeStruct((n, n_e), jnp.float32),
            jax.ShapeDtypeStruct((n, n_e), jnp.float32),
            jax.ShapeDtypeStruct((1, n_e), jnp.float32),
        ],
        compiler_params=pltpu.CompilerParams(
            dimension_semantics=("arbitrary",)),
    )(x, W, b.reshape(1, -1))
    imp = imp.reshape(-1)
    return mask, prob, imp, imp
